# Initial kernel scaffold; baseline (speedup 1.0000x reference)
import jax
import jax.numpy as jnp
from jax.experimental import pallas as pl


def kernel(xyz, features, W0, b0, gamma0, beta0, W1, b1, gamma1, beta1, W2, b2, gamma2, beta2):
    raise NotImplementedError("write your pallas kernel here")


# speedup vs baseline: 3.2198x; 3.2198x over previous
import functools

import jax
import jax.numpy as jnp
import numpy as np
from jax import lax
from jax.experimental import pallas as pl
from jax.experimental.pallas import tpu as pltpu

_B, _N = 8, 4096
_S = 1024
_K = 32
_R2 = float(np.float64(0.2) ** 2)
_CIN = 64
_CPAD = 80
_M = _B * _S * _K


def _fps_body(x_ref, y_ref, z_ref, newxyz_ref):
    B, N, S = _B, _N, _S
    x = x_ref[...]
    y = y_ref[...]
    z = z_ref[...]
    idx2d = lax.broadcasted_iota(jnp.int32, (B, N), 1)

    def body(t, carry):
        dist, far = carry
        onehot = idx2d == far
        cx = jnp.sum(jnp.where(onehot, x, 0.0), axis=1, keepdims=True)
        cy = jnp.sum(jnp.where(onehot, y, 0.0), axis=1, keepdims=True)
        cz = jnp.sum(jnp.where(onehot, z, 0.0), axis=1, keepdims=True)
        newxyz_ref[:, pl.ds(t, 1), :] = jnp.concatenate(
            [cx[:, :, None], cy[:, :, None], cz[:, :, None]], axis=2)
        dx = x - cx
        dy = y - cy
        dz = z - cz
        d = dx * dx + dy * dy + dz * dz
        dist = jnp.minimum(dist, d)
        m = jnp.max(dist, axis=1, keepdims=True)
        nxt = jnp.min(jnp.where(dist == m, idx2d, N), axis=1, keepdims=True)
        return dist, nxt.astype(jnp.int32)

    init = (jnp.full((B, N), 1e10, jnp.float32), jnp.zeros((B, 1), jnp.int32))
    lax.fori_loop(0, S, body, init)


def _fps(x, y, z, *, interpret=False):
    return pl.pallas_call(
        _fps_body,
        out_shape=jax.ShapeDtypeStruct((_B, _S, 3), jnp.float32),
        interpret=interpret,
    )(x, y, z)


_MB = 4096
_SB = _MB // _K
_GRID = _M // _MB


def _p1_body(g_ref, c_ref, w_ref, s_ref, q_ref):
    gc = (g_ref[...].reshape(_SB, _K, _CPAD) - c_ref[...][:, None, :]).reshape(_MB, _CPAD)
    z = jnp.dot(gc, w_ref[...], preferred_element_type=jnp.float32)

    @pl.when(pl.program_id(0) == 0)
    def _():
        s_ref[...] = jnp.zeros_like(s_ref)
        q_ref[...] = jnp.zeros_like(q_ref)

    s_ref[...] += jnp.sum(z, axis=0, keepdims=True)
    q_ref[...] += jnp.sum(z * z, axis=0, keepdims=True)


def _p1(G, C, W0e, *, interpret=False):
    return pl.pallas_call(
        _p1_body,
        grid=(_GRID,),
        in_specs=[
            pl.BlockSpec((_MB, _CPAD), lambda i: (i, 0)),
            pl.BlockSpec((_SB, _CPAD), lambda i: (i, 0)),
            pl.BlockSpec((_CPAD, 64), lambda i: (0, 0)),
        ],
        out_specs=[
            pl.BlockSpec((1, 64), lambda i: (0, 0)),
            pl.BlockSpec((1, 64), lambda i: (0, 0)),
        ],
        out_shape=[
            jax.ShapeDtypeStruct((1, 64), jnp.float32),
            jax.ShapeDtypeStruct((1, 64), jnp.float32),
        ],
        interpret=interpret,
    )(G, C, W0e)


def _p2_body(g_ref, c_ref, w0_ref, a_ref, b_ref, w1_ref, x1_ref, s_ref, q_ref):
    gc = (g_ref[...].reshape(_SB, _K, _CPAD) - c_ref[...][:, None, :]).reshape(_MB, _CPAD)
    z1 = jnp.dot(gc, w0_ref[...], preferred_element_type=jnp.float32)
    x1 = jnp.maximum(a_ref[...] * z1 + b_ref[...], 0.0)
    x1_ref[...] = x1
    z2 = jnp.dot(x1, w1_ref[...], preferred_element_type=jnp.float32)

    @pl.when(pl.program_id(0) == 0)
    def _():
        s_ref[...] = jnp.zeros_like(s_ref)
        q_ref[...] = jnp.zeros_like(q_ref)

    s_ref[...] += jnp.sum(z2, axis=0, keepdims=True)
    q_ref[...] += jnp.sum(z2 * z2, axis=0, keepdims=True)


def _p2(G, C, W0e, a1, c1, W1T, *, interpret=False):
    return pl.pallas_call(
        _p2_body,
        grid=(_GRID,),
        in_specs=[
            pl.BlockSpec((_MB, _CPAD), lambda i: (i, 0)),
            pl.BlockSpec((_SB, _CPAD), lambda i: (i, 0)),
            pl.BlockSpec((_CPAD, 64), lambda i: (0, 0)),
            pl.BlockSpec((1, 64), lambda i: (0, 0)),
            pl.BlockSpec((1, 64), lambda i: (0, 0)),
            pl.BlockSpec((64, 128), lambda i: (0, 0)),
        ],
        out_specs=[
            pl.BlockSpec((_MB, 64), lambda i: (i, 0)),
            pl.BlockSpec((1, 128), lambda i: (0, 0)),
            pl.BlockSpec((1, 128), lambda i: (0, 0)),
        ],
        out_shape=[
            jax.ShapeDtypeStruct((_M, 64), jnp.float32),
            jax.ShapeDtypeStruct((1, 128), jnp.float32),
            jax.ShapeDtypeStruct((1, 128), jnp.float32),
        ],
        interpret=interpret,
    )(G, C, W0e, a1, c1, W1T)


def _p3_body(x1_ref, a_ref, b_ref, w1_ref, w2_ref, x2_ref, s_ref, q_ref):
    z2 = jnp.dot(x1_ref[...], w1_ref[...], preferred_element_type=jnp.float32)
    x2 = jnp.maximum(a_ref[...] * z2 + b_ref[...], 0.0)
    x2_ref[...] = x2
    z3 = jnp.dot(x2, w2_ref[...], preferred_element_type=jnp.float32)

    @pl.when(pl.program_id(0) == 0)
    def _():
        s_ref[...] = jnp.zeros_like(s_ref)
        q_ref[...] = jnp.zeros_like(q_ref)

    s_ref[...] += jnp.sum(z3, axis=0, keepdims=True)
    q_ref[...] += jnp.sum(z3 * z3, axis=0, keepdims=True)


def _p3(X1, a2, c2, W1T, W2T, *, interpret=False):
    return pl.pallas_call(
        _p3_body,
        grid=(_GRID,),
        in_specs=[
            pl.BlockSpec((_MB, 64), lambda i: (i, 0)),
            pl.BlockSpec((1, 128), lambda i: (0, 0)),
            pl.BlockSpec((1, 128), lambda i: (0, 0)),
            pl.BlockSpec((64, 128), lambda i: (0, 0)),
            pl.BlockSpec((128, 256), lambda i: (0, 0)),
        ],
        out_specs=[
            pl.BlockSpec((_MB, 128), lambda i: (i, 0)),
            pl.BlockSpec((1, 256), lambda i: (0, 0)),
            pl.BlockSpec((1, 256), lambda i: (0, 0)),
        ],
        out_shape=[
            jax.ShapeDtypeStruct((_M, 128), jnp.float32),
            jax.ShapeDtypeStruct((1, 256), jnp.float32),
            jax.ShapeDtypeStruct((1, 256), jnp.float32),
        ],
        interpret=interpret,
    )(X1, a2, c2, W1T, W2T)


def _p4_body(x2_ref, a_ref, b_ref, w2_ref, out_ref):
    z3 = jnp.dot(x2_ref[...], w2_ref[...], preferred_element_type=jnp.float32)
    z3r = z3.reshape(_SB, _K, 256)
    zmax = jnp.max(z3r, axis=1)
    zmin = jnp.min(z3r, axis=1)
    a = a_ref[...]
    zsel = jnp.where(a > 0, zmax, zmin)
    out_ref[...] = jnp.maximum(a * zsel + b_ref[...], 0.0)


def _p4(X2, a3, c3, W2T, *, interpret=False):
    return pl.pallas_call(
        _p4_body,
        grid=(_GRID,),
        in_specs=[
            pl.BlockSpec((_MB, 128), lambda i: (i, 0)),
            pl.BlockSpec((1, 256), lambda i: (0, 0)),
            pl.BlockSpec((1, 256), lambda i: (0, 0)),
            pl.BlockSpec((128, 256), lambda i: (0, 0)),
        ],
        out_specs=pl.BlockSpec((_SB, 256), lambda i: (i, 0)),
        out_shape=jax.ShapeDtypeStruct((_B * _S, 256), jnp.float32),
        interpret=interpret,
    )(X2, a3, c3, W2T)


def _bn_fold(s, q, gamma, beta):
    mu = s / _M
    var = q / _M - mu * mu
    a = gamma[None, :] / jnp.sqrt(var + 1e-5)
    c = beta[None, :] - a * mu
    return a, c


def _ball_query_xla(xyz, new_xyz):
    sqr = jnp.sum((new_xyz[:, :, None, :] - xyz[:, None, :, :]) ** 2, axis=-1)
    ar = jnp.arange(_N, dtype=jnp.int32)
    idx = jnp.where(sqr < _R2, ar[None, None, :], _N)
    idx = jnp.sort(idx, axis=-1)[:, :, :_K]
    nearest = jnp.argmin(sqr, axis=-1).astype(jnp.int32)[:, :, None]
    first = idx[:, :, :1]
    first = jnp.where(first == _N, nearest, first)
    idx = jnp.where(idx == _N, first, idx)
    boff = jnp.arange(_B, dtype=jnp.int32)[:, None, None] * _N
    return (idx + boff).reshape(_M)


def kernel(xyz, features, W0, b0, gamma0, beta0, W1, b1, gamma1, beta1,
           W2, b2, gamma2, beta2):
    x = xyz[..., 0]
    y = xyz[..., 1]
    z = xyz[..., 2]
    new_xyz = _fps(x, y, z)

    gidx = _ball_query_xla(xyz, new_xyz)

    table = jnp.concatenate(
        [xyz, jnp.zeros((_B, _N, _CPAD - 3 - _CIN), jnp.float32), features],
        axis=-1).reshape(_B * _N, _CPAD)
    G = table[gidx]

    C = jnp.concatenate(
        [new_xyz, jnp.zeros((_B, _S, _CPAD - 3), jnp.float32)],
        axis=-1).reshape(_B * _S, _CPAD)

    W0e = jnp.zeros((_CPAD, 64), jnp.float32)
    W0e = W0e.at[0:3, :].set(W0[:, 0:3].T)
    W0e = W0e.at[16:_CPAD, :].set(W0[:, 3:67].T)
    W1T = W1.T
    W2T = W2.T

    s1, q1 = _p1(G, C, W0e)
    a1, c1 = _bn_fold(s1, q1, gamma0, beta0)
    X1, s2, q2 = _p2(G, C, W0e, a1, c1, W1T)
    a2, c2 = _bn_fold(s2, q2, gamma1, beta1)
    X2, s3, q3 = _p3(X1, a2, c2, W1T, W2T)
    a3, c3 = _bn_fold(s3, q3, gamma2, beta2)
    out = _p4(X2, a3, c3, W2T)
    return new_xyz, out.reshape(_B, _S, 256)
